# baseline (speedup 1.0000x reference)
import functools

import jax
import jax.numpy as jnp
from jax import lax
from jax.experimental import pallas as pl
from jax.experimental.pallas import tpu as pltpu
from jax.experimental.pallas import tpu_sc as plsc

_B = 4096
_NW = 32
_NBT = _B // 128
_NSLOT = 12
_ROW = 848
_NCT = _ROW // 8

_NP = 2
_HB = _B // _NP

_GW = 896
_GC_SP = 0
_GC_MV = 128
_GC_IT = 640
_GC_AB = 768
_GCOLS = (_GC_SP, _GC_MV, _GC_MV + 128, _GC_MV + 256, _GC_MV + 384,
          _GC_IT, _GC_AB)

_MA_DIM = 32
_PA_DIM = 48

_BTG = 16


def _gather_body(idx_h, sp_t, mv_t, it_t, ab_t, g_h, idx_v, *refs):
    nbw = _HB // _NW
    cb = nbw // 2
    bufs = (refs[0:7], refs[7:14])
    sems = (refs[14], refs[15])
    wid = lax.axis_index("s") * 2 + lax.axis_index("c")
    b0 = wid * nbw
    wpt = 128 // nbw
    boff = (wid % wpt) * nbw
    pltpu.sync_copy(idx_h.at[:, pl.ds((wid // wpt) * 128, 128)], idx_v)

    def drain(par, gslot, ghalf):
        rows = pl.ds(gslot * _HB + b0 + ghalf * cb, cb)
        for j in range(7):
            pltpu.make_async_copy(
                bufs[par][j], g_h.at[rows, pl.ds(_GCOLS[j], 128)],
                sems[par]).wait()

    for c in range(_NSLOT * 2):
        slot, half = c // 2, c % 2
        par = c % 2
        s, p = slot // 6, slot % 6
        q_sp = p * 2 + s
        if c >= 2:
            pc = c - 2
            drain(par, pc // 2, pc % 2)
        bsl = pl.ds(boff + half * cb, cb)
        tabs = (sp_t, mv_t, mv_t, mv_t, mv_t, it_t, ab_t)
        qrows = (q_sp, 36 + slot * 4, 37 + slot * 4, 38 + slot * 4,
                 39 + slot * 4, 12 + q_sp, 24 + q_sp)
        gathers = [
            pltpu.async_copy(tabs[j].at[idx_v.at[qrows[j], bsl]],
                             bufs[par][j], sems[par])
            for j in range(7)
        ]
        for g in gathers:
            g.wait()
        rows = pl.ds(slot * _HB + b0 + half * cb, cb)
        for j in range(7):
            pltpu.async_copy(bufs[par][j],
                             g_h.at[rows, pl.ds(_GCOLS[j], 128)], sems[par])
    drain(0, 11, 0)
    drain(1, 11, 1)


@jax.jit
def _gather(idx_half, sp_t, mv_t, it_tp, ab_tp):
    nbw = _HB // _NW
    cb = nbw // 2
    mesh = plsc.VectorSubcoreMesh(core_axis_name="c", subcore_axis_name="s")
    return pl.kernel(
        _gather_body,
        out_type=jax.ShapeDtypeStruct((_NSLOT * _HB, _GW), jnp.float32),
        mesh=mesh,
        scratch_types=[pltpu.VMEM((84, 128), jnp.int32)]
        + [pltpu.VMEM((cb, 128), jnp.float32) for _ in range(14)]
        + [pltpu.SemaphoreType.DMA, pltpu.SemaphoreType.DMA],
    )(idx_half, sp_t, mv_t, it_tp, ab_tp)


def _asm_body(g_ref, ma_ref, pa_ref, *rest):
    o_ref = rest[-1]
    for k in range(_BTG):
        x = g_ref[pl.ds(k * 128, 128), :]
        y = jnp.transpose(x)
        o_ref[0, 0, pl.ds(0, 16), k] = y[0:128].reshape(16, 8, 128)
        o_ref[0, 0, pl.ds(16, 8), k] = y[640:704].reshape(8, 8, 128)
        o_ref[0, 0, pl.ds(24, 8), k] = y[768:832].reshape(8, 8, 128)
        o_ref[0, 0, pl.ds(32, 64), k] = y[128:640].reshape(64, 8, 128)
        o_ref[0, 0, pl.ds(96, 4), k] = ma_ref[0, 0, :, k]
        o_ref[0, 0, pl.ds(100, 6), k] = pa_ref[0, 0, :, k]


_HBT = _HB // 128 // _BTG


def _mk_specs(hh):
    in_specs = [
        pl.BlockSpec((128 * _BTG, _GW), lambda slot, bt: (slot * _HBT + bt, 0)),
        pl.BlockSpec((1, 1, 4, _BTG, 8, 128),
                     lambda slot, bt: (slot // 6, slot % 6, 0, hh * _HBT + bt, 0, 0)),
        pl.BlockSpec((1, 1, 6, _BTG, 8, 128),
                     lambda slot, bt: (slot // 6, slot % 6, 0, hh * _HBT + bt, 0, 0)),
    ]
    out_spec = pl.BlockSpec(
        (1, 1, _NCT, _BTG, 8, 128),
        lambda slot, bt: (slot // 6, slot % 6, 0, hh * _HBT + bt, 0, 0))
    return in_specs, out_spec


_OUT6 = jax.ShapeDtypeStruct((2, 6, _NCT, _NBT, 8, 128), jnp.float32)


@jax.jit
def _assemble_first(g, ma_l, pa_l):
    in_specs, out_spec = _mk_specs(0)
    return pl.pallas_call(
        _asm_body, grid=(_NSLOT, _HBT), in_specs=in_specs,
        out_specs=out_spec, out_shape=_OUT6,
    )(g, ma_l, pa_l)


@functools.partial(jax.jit, static_argnames=("part",))
def _assemble_next(g, ma_l, pa_l, prev, part):
    in_specs, out_spec = _mk_specs(part)
    in_specs.append(pl.BlockSpec(memory_space=pl.ANY))
    return pl.pallas_call(
        _asm_body, grid=(_NSLOT, _HBT), in_specs=in_specs,
        out_specs=out_spec, out_shape=_OUT6,
        input_output_aliases={3: 0},
    )(g, ma_l, pa_l, prev)


def kernel(fields, sides, species, moves, items, abilities, move_attributes,
           pokemon_attributes, species_table, item_table, ability_table,
           move_table):
    sp_i = jnp.transpose(species, (2, 1, 0)).reshape(12, _B)
    it_i = jnp.transpose(items, (2, 1, 0)).reshape(12, _B)
    ab_i = jnp.transpose(abilities, (2, 1, 0)).reshape(12, _B)
    mv_i = jnp.transpose(moves, (1, 2, 3, 0)).reshape(48, _B)
    idx_all = jnp.concatenate([sp_i, it_i, ab_i, mv_i], axis=0)
    it_tp = jnp.pad(item_table, ((0, 0), (0, 64)))
    ab_tp = jnp.pad(ability_table, ((0, 0), (0, 64)))
    ma_l = (move_attributes.transpose(1, 2, 3, 0, 4)
            .reshape(2, 6, 4, _NBT, 128, 8).transpose(0, 1, 2, 3, 5, 4))
    pa_l = (pokemon_attributes.transpose(1, 2, 3, 0)
            .reshape(2, 6, 6, 8, _NBT, 128).transpose(0, 1, 2, 4, 3, 5))
    o = None
    for part in range(_NP):
        idx_p = lax.slice_in_dim(idx_all, part * _HB, (part + 1) * _HB, axis=1)
        g = _gather(idx_p, species_table, move_table, it_tp, ab_tp)
        if part == 0:
            o = _assemble_first(g, ma_l, pa_l)
        else:
            o = _assemble_next(g, ma_l, pa_l, o, part=part)
    out = o.transpose(3, 5, 0, 1, 2, 4).reshape(_B, 2, 6, _ROW)
    return (fields, sides, out)


# speedup vs baseline: 1.0060x; 1.0060x over previous
import functools

import jax
import jax.numpy as jnp
from jax import lax
from jax.experimental import pallas as pl
from jax.experimental.pallas import tpu as pltpu
from jax.experimental.pallas import tpu_sc as plsc

_B = 4096
_NW = 32
_NBT = _B // 128
_NSLOT = 12
_ROW = 848
_NCT = _ROW // 8

_NP = 2
_HB = _B // _NP

_GW = 896
_GC_SP = 0
_GC_MV = 128
_GC_IT = 640
_GC_AB = 768
_GCOLS = (_GC_SP, _GC_MV, _GC_MV + 128, _GC_MV + 256, _GC_MV + 384,
          _GC_IT, _GC_AB)

_MA_DIM = 32
_PA_DIM = 48

_BTG = 8


def _gather_body(idx_h, sp_t, mv_t, it_t, ab_t, g_h, idx_v, *refs):
    nbw = _HB // _NW
    cb = nbw // 2
    bufs = (refs[0:7], refs[7:14])
    sems = (refs[14], refs[15])
    wid = lax.axis_index("s") * 2 + lax.axis_index("c")
    b0 = wid * nbw
    wpt = 128 // nbw
    boff = (wid % wpt) * nbw
    pltpu.sync_copy(idx_h.at[:, pl.ds((wid // wpt) * 128, 128)], idx_v)

    def drain(par, gslot, ghalf):
        rows = pl.ds(gslot * _HB + b0 + ghalf * cb, cb)
        for j in range(7):
            pltpu.make_async_copy(
                bufs[par][j], g_h.at[rows, pl.ds(_GCOLS[j], 128)],
                sems[par]).wait()

    for c in range(_NSLOT * 2):
        slot, half = c // 2, c % 2
        par = c % 2
        s, p = slot // 6, slot % 6
        q_sp = p * 2 + s
        if c >= 2:
            pc = c - 2
            drain(par, pc // 2, pc % 2)
        bsl = pl.ds(boff + half * cb, cb)
        tabs = (sp_t, mv_t, mv_t, mv_t, mv_t, it_t, ab_t)
        qrows = (q_sp, 36 + slot * 4, 37 + slot * 4, 38 + slot * 4,
                 39 + slot * 4, 12 + q_sp, 24 + q_sp)
        gathers = [
            pltpu.async_copy(tabs[j].at[idx_v.at[qrows[j], bsl]],
                             bufs[par][j], sems[par])
            for j in range(7)
        ]
        for g in gathers:
            g.wait()
        rows = pl.ds(slot * _HB + b0 + half * cb, cb)
        for j in range(7):
            pltpu.async_copy(bufs[par][j],
                             g_h.at[rows, pl.ds(_GCOLS[j], 128)], sems[par])
    drain(0, 11, 0)
    drain(1, 11, 1)


@jax.jit
def _gather(idx_half, sp_t, mv_t, it_tp, ab_tp):
    nbw = _HB // _NW
    cb = nbw // 2
    mesh = plsc.VectorSubcoreMesh(core_axis_name="c", subcore_axis_name="s")
    return pl.kernel(
        _gather_body,
        out_type=jax.ShapeDtypeStruct((_NSLOT * _HB, _GW), jnp.float32),
        mesh=mesh,
        scratch_types=[pltpu.VMEM((84, 128), jnp.int32)]
        + [pltpu.VMEM((cb, 128), jnp.float32) for _ in range(14)]
        + [pltpu.SemaphoreType.DMA, pltpu.SemaphoreType.DMA],
    )(idx_half, sp_t, mv_t, it_tp, ab_tp)


def _asm_body(g_ref, ma_ref, pa_ref, *rest):
    o_ref = rest[-1]
    for k in range(_BTG):
        x = g_ref[pl.ds(k * 128, 128), :]
        y = jnp.transpose(x)
        o_ref[0, 0, pl.ds(0, 16), k] = y[0:128].reshape(16, 8, 128)
        o_ref[0, 0, pl.ds(16, 8), k] = y[640:704].reshape(8, 8, 128)
        o_ref[0, 0, pl.ds(24, 8), k] = y[768:832].reshape(8, 8, 128)
        o_ref[0, 0, pl.ds(32, 64), k] = y[128:640].reshape(64, 8, 128)
        o_ref[0, 0, pl.ds(96, 4), k] = ma_ref[0, 0, :, k]
        o_ref[0, 0, pl.ds(100, 6), k] = pa_ref[0, 0, :, k]


_HBT = _HB // 128 // _BTG


def _mk_specs(hh):
    in_specs = [
        pl.BlockSpec((128 * _BTG, _GW), lambda slot, bt: (slot * _HBT + bt, 0)),
        pl.BlockSpec((1, 1, 4, _BTG, 8, 128),
                     lambda slot, bt: (slot // 6, slot % 6, 0, hh * _HBT + bt, 0, 0)),
        pl.BlockSpec((1, 1, 6, _BTG, 8, 128),
                     lambda slot, bt: (slot // 6, slot % 6, 0, hh * _HBT + bt, 0, 0)),
    ]
    out_spec = pl.BlockSpec(
        (1, 1, _NCT, _BTG, 8, 128),
        lambda slot, bt: (slot // 6, slot % 6, 0, hh * _HBT + bt, 0, 0))
    return in_specs, out_spec


_OUT6 = jax.ShapeDtypeStruct((2, 6, _NCT, _NBT, 8, 128), jnp.float32)


@jax.jit
def _assemble_first(g, ma_l, pa_l):
    in_specs, out_spec = _mk_specs(0)
    return pl.pallas_call(
        _asm_body, grid=(_NSLOT, _HBT), in_specs=in_specs,
        out_specs=out_spec, out_shape=_OUT6,
    )(g, ma_l, pa_l)


@functools.partial(jax.jit, static_argnames=("part",))
def _assemble_next(g, ma_l, pa_l, prev, part):
    in_specs, out_spec = _mk_specs(part)
    in_specs.append(pl.BlockSpec(memory_space=pl.ANY))
    return pl.pallas_call(
        _asm_body, grid=(_NSLOT, _HBT), in_specs=in_specs,
        out_specs=out_spec, out_shape=_OUT6,
        input_output_aliases={3: 0},
    )(g, ma_l, pa_l, prev)


def kernel(fields, sides, species, moves, items, abilities, move_attributes,
           pokemon_attributes, species_table, item_table, ability_table,
           move_table):
    sp_i = jnp.transpose(species, (2, 1, 0)).reshape(12, _B)
    it_i = jnp.transpose(items, (2, 1, 0)).reshape(12, _B)
    ab_i = jnp.transpose(abilities, (2, 1, 0)).reshape(12, _B)
    mv_i = jnp.transpose(moves, (1, 2, 3, 0)).reshape(48, _B)
    idx_all = jnp.concatenate([sp_i, it_i, ab_i, mv_i], axis=0)
    it_tp = jnp.pad(item_table, ((0, 0), (0, 64)))
    ab_tp = jnp.pad(ability_table, ((0, 0), (0, 64)))
    ma_l = (move_attributes.transpose(1, 2, 3, 0, 4)
            .reshape(2, 6, 4, _NBT, 128, 8).transpose(0, 1, 2, 3, 5, 4))
    pa_l = (pokemon_attributes.transpose(1, 2, 3, 0)
            .reshape(2, 6, 6, 8, _NBT, 128).transpose(0, 1, 2, 4, 3, 5))
    o = None
    for part in range(_NP):
        idx_p = lax.slice_in_dim(idx_all, part * _HB, (part + 1) * _HB, axis=1)
        g = _gather(idx_p, species_table, move_table, it_tp, ab_tp)
        if part == 0:
            o = _assemble_first(g, ma_l, pa_l)
        else:
            o = _assemble_next(g, ma_l, pa_l, o, part=part)
    out = o.transpose(3, 5, 0, 1, 2, 4).reshape(_B, 2, 6, _ROW)
    return (fields, sides, out)


# speedup vs baseline: 1.0129x; 1.0068x over previous
import functools

import jax
import jax.numpy as jnp
from jax import lax
from jax.experimental import pallas as pl
from jax.experimental.pallas import tpu as pltpu
from jax.experimental.pallas import tpu_sc as plsc

_B = 4096
_NW = 32
_NBT = _B // 128
_NSLOT = 12
_ROW = 848
_NCT = _ROW // 8

_NP = 2
_HB = _B // _NP

_GW = 896
_GC_SP = 0
_GC_MV = 128
_GC_IT = 640
_GC_AB = 768
_GCOLS = (_GC_SP, _GC_MV, _GC_MV + 128, _GC_MV + 256, _GC_MV + 384,
          _GC_IT, _GC_AB)

_MA_DIM = 32
_PA_DIM = 48

_BTG = 8


def _gather_body(part, idx_h, sp_t, mv_t, it_t, ab_t, g_h, idx_v, *refs):
    nbw = _HB // _NW
    cb = nbw // 2
    bufs = (refs[0:7], refs[7:14])
    sems = (refs[14], refs[15])
    wid = lax.axis_index("s") * 2 + lax.axis_index("c")
    b0 = wid * nbw
    wpt = 128 // nbw
    boff = (wid % wpt) * nbw
    pltpu.sync_copy(
        idx_h.at[:, pl.ds(part * _HB + (wid // wpt) * 128, 128)], idx_v)

    def drain(par, gslot, ghalf):
        rows = pl.ds(gslot * _HB + b0 + ghalf * cb, cb)
        for j in range(7):
            pltpu.make_async_copy(
                bufs[par][j], g_h.at[rows, pl.ds(_GCOLS[j], 128)],
                sems[par]).wait()

    for c in range(_NSLOT * 2):
        slot, half = c // 2, c % 2
        par = c % 2
        s, p = slot // 6, slot % 6
        q_sp = p * 2 + s
        if c >= 2:
            pc = c - 2
            drain(par, pc // 2, pc % 2)
        bsl = pl.ds(boff + half * cb, cb)
        tabs = (sp_t, mv_t, mv_t, mv_t, mv_t, it_t, ab_t)
        qrows = (q_sp, 36 + slot * 4, 37 + slot * 4, 38 + slot * 4,
                 39 + slot * 4, 12 + q_sp, 24 + q_sp)
        gathers = [
            pltpu.async_copy(tabs[j].at[idx_v.at[qrows[j], bsl]],
                             bufs[par][j], sems[par])
            for j in range(7)
        ]
        for g in gathers:
            g.wait()
        rows = pl.ds(slot * _HB + b0 + half * cb, cb)
        for j in range(7):
            pltpu.async_copy(bufs[par][j],
                             g_h.at[rows, pl.ds(_GCOLS[j], 128)], sems[par])
    drain(0, 11, 0)
    drain(1, 11, 1)


@functools.partial(jax.jit, static_argnames=("part",))
def _gather(idx_all, sp_t, mv_t, it_tp, ab_tp, part):
    nbw = _HB // _NW
    cb = nbw // 2
    mesh = plsc.VectorSubcoreMesh(core_axis_name="c", subcore_axis_name="s")
    return pl.kernel(
        functools.partial(_gather_body, part),
        out_type=jax.ShapeDtypeStruct((_NSLOT * _HB, _GW), jnp.float32),
        mesh=mesh,
        scratch_types=[pltpu.VMEM((84, 128), jnp.int32)]
        + [pltpu.VMEM((cb, 128), jnp.float32) for _ in range(14)]
        + [pltpu.SemaphoreType.DMA, pltpu.SemaphoreType.DMA],
    )(idx_all, sp_t, mv_t, it_tp, ab_tp)


def _asm_body(g_ref, ma_ref, pa_ref, *rest):
    o_ref = rest[-1]
    for k in range(_BTG):
        x = g_ref[pl.ds(k * 128, 128), :]
        y = jnp.transpose(x)
        o_ref[0, 0, pl.ds(0, 16), k] = y[0:128].reshape(16, 8, 128)
        o_ref[0, 0, pl.ds(16, 8), k] = y[640:704].reshape(8, 8, 128)
        o_ref[0, 0, pl.ds(24, 8), k] = y[768:832].reshape(8, 8, 128)
        o_ref[0, 0, pl.ds(32, 64), k] = y[128:640].reshape(64, 8, 128)
        o_ref[0, 0, pl.ds(96, 4), k] = ma_ref[0, 0, :, k]
        o_ref[0, 0, pl.ds(100, 6), k] = pa_ref[0, 0, :, k]


_HBT = _HB // 128 // _BTG


def _mk_specs(hh):
    in_specs = [
        pl.BlockSpec((128 * _BTG, _GW), lambda slot, bt: (slot * _HBT + bt, 0)),
        pl.BlockSpec((1, 1, 4, _BTG, 8, 128),
                     lambda slot, bt: (slot // 6, slot % 6, 0, hh * _HBT + bt, 0, 0)),
        pl.BlockSpec((1, 1, 6, _BTG, 8, 128),
                     lambda slot, bt: (slot // 6, slot % 6, 0, hh * _HBT + bt, 0, 0)),
    ]
    out_spec = pl.BlockSpec(
        (1, 1, _NCT, _BTG, 8, 128),
        lambda slot, bt: (slot // 6, slot % 6, 0, hh * _HBT + bt, 0, 0))
    return in_specs, out_spec


_OUT6 = jax.ShapeDtypeStruct((2, 6, _NCT, _NBT, 8, 128), jnp.float32)


@jax.jit
def _assemble_first(g, ma_l, pa_l):
    in_specs, out_spec = _mk_specs(0)
    return pl.pallas_call(
        _asm_body, grid=(_NSLOT, _HBT), in_specs=in_specs,
        out_specs=out_spec, out_shape=_OUT6,
    )(g, ma_l, pa_l)


@functools.partial(jax.jit, static_argnames=("part",))
def _assemble_next(g, ma_l, pa_l, prev, part):
    in_specs, out_spec = _mk_specs(part)
    in_specs.append(pl.BlockSpec(memory_space=pl.ANY))
    return pl.pallas_call(
        _asm_body, grid=(_NSLOT, _HBT), in_specs=in_specs,
        out_specs=out_spec, out_shape=_OUT6,
        input_output_aliases={3: 0},
    )(g, ma_l, pa_l, prev)


def kernel(fields, sides, species, moves, items, abilities, move_attributes,
           pokemon_attributes, species_table, item_table, ability_table,
           move_table):
    sp_i = jnp.transpose(species, (2, 1, 0)).reshape(12, _B)
    it_i = jnp.transpose(items, (2, 1, 0)).reshape(12, _B)
    ab_i = jnp.transpose(abilities, (2, 1, 0)).reshape(12, _B)
    mv_i = jnp.transpose(moves, (1, 2, 3, 0)).reshape(48, _B)
    idx_all = jnp.concatenate([sp_i, it_i, ab_i, mv_i], axis=0)
    it_tp = jnp.pad(item_table, ((0, 0), (0, 64)))
    ab_tp = jnp.pad(ability_table, ((0, 0), (0, 64)))
    ma_l = (move_attributes.transpose(1, 2, 3, 0, 4)
            .reshape(2, 6, 4, _NBT, 128, 8).transpose(0, 1, 2, 3, 5, 4))
    pa_l = (pokemon_attributes.transpose(1, 2, 3, 0)
            .reshape(2, 6, 6, 8, _NBT, 128).transpose(0, 1, 2, 4, 3, 5))
    o = None
    for part in range(_NP):
        g = _gather(idx_all, species_table, move_table, it_tp, ab_tp, part=part)
        if part == 0:
            o = _assemble_first(g, ma_l, pa_l)
        else:
            o = _assemble_next(g, ma_l, pa_l, o, part=part)
    out = o.transpose(3, 5, 0, 1, 2, 4).reshape(_B, 2, 6, _ROW)
    return (fields, sides, out)
